# baseline (speedup 1.0000x reference)
import functools

import jax
import jax.numpy as jnp
from jax import lax
from jax.experimental import pallas as pl
from jax.experimental.pallas import tpu as pltpu
from jax.experimental.pallas import tpu_sc as plsc

NC = 2
NS = 16
NW = NC * NS
L = 16
STRIP = 128
RING = 4
SIDES = 3


@functools.partial(jax.jit, static_argnames=("B", "D"))
def _gmf(uidx, iidx, ut_t, it_t, w_flat, b_vec, *, B, D):
    bpw = B // NW
    n_groups = bpw // L

    mesh = plsc.VectorSubcoreMesh(
        core_axis_name="c", subcore_axis_name="s",
        num_cores=NC, num_subcores=NS,
    )

    @functools.partial(
        pl.kernel,
        out_type=jax.ShapeDtypeStruct((B,), jnp.float32),
        mesh=mesh,
        compiler_params=pltpu.CompilerParams(
            needs_layout_passes=False, use_tc_tiling_on_sc=True),
        scratch_types=[
            pltpu.VMEM((bpw,), jnp.int32),
            pltpu.VMEM((bpw,), jnp.int32),
            pltpu.VMEM((SIDES * RING, D, STRIP), jnp.float32),
            pltpu.VMEM((SIDES * RING, D, STRIP), jnp.float32),
            pltpu.VMEM((D,), jnp.float32),
            pltpu.VMEM((L,), jnp.float32),
            pltpu.VMEM((bpw,), jnp.float32),
        ] + [pltpu.SemaphoreType.DMA] * SIDES,
    )
    def k(uidx_hbm, iidx_hbm, ut_hbm, it_hbm, w_hbm, b_hbm, out_hbm,
          ui_v, ii_v, us_v, is_v, w_v, b_v, out_v, *sems):
        wid = lax.axis_index("s") * NC + lax.axis_index("c")
        base = wid * bpw
        pltpu.sync_copy(uidx_hbm.at[pl.ds(base, bpw)], ui_v)
        pltpu.sync_copy(iidx_hbm.at[pl.ds(base, bpw)], ii_v)
        pltpu.sync_copy(w_hbm, w_v)
        pltpu.sync_copy(b_hbm, b_v)

        lane = lax.iota(jnp.int32, L)
        wlo = w_v[pl.ds(0, L)]
        whi = w_v[pl.ds(L, L)]
        bias = b_v[...]

        n_batches = L // RING

        def group(g, carry):
            uv = ui_v[pl.ds(g * L, L)]
            iv = ii_v[pl.ds(g * L, L)]

            def fire(jb):
                side = jb % SIDES
                cps = []
                for j in range(RING):
                    e = jb * RING + j
                    slot = side * RING + j
                    u0 = pl.multiple_of((uv[e] >> 7) * STRIP, STRIP)
                    i0 = pl.multiple_of((iv[e] >> 7) * STRIP, STRIP)
                    cps.append(pltpu.async_copy(
                        ut_hbm.at[:, pl.ds(u0, STRIP)], us_v.at[slot], sems[side]))
                    cps.append(pltpu.async_copy(
                        it_hbm.at[:, pl.ds(i0, STRIP)], is_v.at[slot], sems[side]))
                return cps

            acc = bias
            pending = {0: fire(0), 1: fire(1)}
            for jb in range(n_batches):
                if jb + 2 < n_batches:
                    pending[jb + 2] = fire(jb + 2)
                for cp in pending.pop(jb):
                    cp.wait()
                side = jb % SIDES
                for j in range(RING):
                    e = jb * RING + j
                    slot = side * RING + j
                    cu = jnp.full((L,), 0, jnp.int32) + (uv[e] & (STRIP - 1))
                    ci = jnp.full((L,), 0, jnp.int32) + (iv[e] & (STRIP - 1))
                    u_lo = plsc.load_gather(us_v.at[slot], [lane, cu])
                    u_hi = plsc.load_gather(us_v.at[slot], [lane + L, cu])
                    i_lo = plsc.load_gather(is_v.at[slot], [lane, ci])
                    i_hi = plsc.load_gather(is_v.at[slot], [lane + L, ci])
                    s = jnp.sum(u_lo * i_lo * wlo + u_hi * i_hi * whi)
                    acc = jnp.where(lane == e, acc + s, acc)
            out_v[pl.ds(g * L, L)] = 1.0 / (1.0 + jnp.exp(-acc))
            return carry

        lax.fori_loop(0, n_groups, group, 0)
        pltpu.sync_copy(out_v, out_hbm.at[pl.ds(base, bpw)])

    return k(uidx, iidx, ut_t, it_t, w_flat, b_vec)


def kernel(user_indices, item_indices, user_table, item_table, W, b):
    B = user_indices.shape[0]
    D = user_table.shape[1]
    w_flat = W.reshape(D).astype(jnp.float32)
    b_vec = jnp.broadcast_to(b.reshape(1), (L,)).astype(jnp.float32)
    out = _gmf(user_indices.astype(jnp.int32), item_indices.astype(jnp.int32),
               user_table.T, item_table.T, w_flat, b_vec, B=B, D=D)
    return out.reshape(B, 1)


# speedup vs baseline: 1.0892x; 1.0892x over previous
import functools

import jax
import jax.numpy as jnp
from jax import lax
from jax.experimental import pallas as pl
from jax.experimental.pallas import tpu as pltpu
from jax.experimental.pallas import tpu_sc as plsc

NC = 2
NS = 16
NW = NC * NS
L = 16
STRIP = 128
RING = 4
SIDES = 2


@functools.partial(jax.jit, static_argnames=("B", "D"))
def _gmf(uidx, iidx, ut_t, it_t, w_flat, b_vec, *, B, D):
    bpw = B // NW
    n_groups = bpw // L

    mesh = plsc.VectorSubcoreMesh(
        core_axis_name="c", subcore_axis_name="s",
        num_cores=NC, num_subcores=NS,
    )

    @functools.partial(
        pl.kernel,
        out_type=jax.ShapeDtypeStruct((B,), jnp.float32),
        mesh=mesh,
        compiler_params=pltpu.CompilerParams(
            needs_layout_passes=False, use_tc_tiling_on_sc=True),
        scratch_types=[
            pltpu.VMEM((bpw,), jnp.int32),
            pltpu.VMEM((bpw,), jnp.int32),
            pltpu.VMEM((SIDES * RING, D, STRIP), jnp.float32),
            pltpu.VMEM((SIDES * RING, D, STRIP), jnp.float32),
            pltpu.VMEM((D,), jnp.float32),
            pltpu.VMEM((L,), jnp.float32),
            pltpu.VMEM((bpw,), jnp.float32),
        ] + [pltpu.SemaphoreType.DMA] * SIDES,
    )
    def k(uidx_hbm, iidx_hbm, ut_hbm, it_hbm, w_hbm, b_hbm, out_hbm,
          ui_v, ii_v, us_v, is_v, w_v, b_v, out_v, *sems):
        wid = lax.axis_index("s") * NC + lax.axis_index("c")
        base = wid * bpw
        pltpu.sync_copy(uidx_hbm.at[pl.ds(base, bpw)], ui_v)
        pltpu.sync_copy(iidx_hbm.at[pl.ds(base, bpw)], ii_v)
        pltpu.sync_copy(w_hbm, w_v)
        pltpu.sync_copy(b_hbm, b_v)

        lane = lax.iota(jnp.int32, L)
        wlo = w_v[pl.ds(0, L)]
        whi = w_v[pl.ds(L, L)]
        bias = b_v[...]

        n_batches = L // RING

        def group(g, carry):
            uv = ui_v[pl.ds(g * L, L)]
            iv = ii_v[pl.ds(g * L, L)]

            def fire(jb):
                side = jb % SIDES
                cps = []
                for j in range(RING):
                    e = jb * RING + j
                    slot = side * RING + j
                    u0 = pl.multiple_of((uv[e] >> 7) * STRIP, STRIP)
                    i0 = pl.multiple_of((iv[e] >> 7) * STRIP, STRIP)
                    cps.append(pltpu.async_copy(
                        ut_hbm.at[:, pl.ds(u0, STRIP)], us_v.at[slot], sems[side]))
                    cps.append(pltpu.async_copy(
                        it_hbm.at[:, pl.ds(i0, STRIP)], is_v.at[slot], sems[side]))
                return cps

            acc = bias
            pending = {0: fire(0)}
            for jb in range(n_batches):
                if jb + 1 < n_batches:
                    pending[jb + 1] = fire(jb + 1)
                for cp in pending.pop(jb):
                    cp.wait()
                side = jb % SIDES
                for j in range(RING):
                    e = jb * RING + j
                    slot = side * RING + j
                    cu = jnp.full((L,), 0, jnp.int32) + (uv[e] & (STRIP - 1))
                    ci = jnp.full((L,), 0, jnp.int32) + (iv[e] & (STRIP - 1))
                    u_lo = plsc.load_gather(us_v.at[slot], [lane, cu])
                    u_hi = plsc.load_gather(us_v.at[slot], [lane + L, cu])
                    i_lo = plsc.load_gather(is_v.at[slot], [lane, ci])
                    i_hi = plsc.load_gather(is_v.at[slot], [lane + L, ci])
                    s = jnp.sum(u_lo * i_lo * wlo + u_hi * i_hi * whi)
                    acc = jnp.where(lane == e, acc + s, acc)
            out_v[pl.ds(g * L, L)] = 1.0 / (1.0 + jnp.exp(-acc))
            return carry

        lax.fori_loop(0, n_groups, group, 0)
        pltpu.sync_copy(out_v, out_hbm.at[pl.ds(base, bpw)])

    return k(uidx, iidx, ut_t, it_t, w_flat, b_vec)


def kernel(user_indices, item_indices, user_table, item_table, W, b):
    B = user_indices.shape[0]
    D = user_table.shape[1]
    w_flat = W.reshape(D).astype(jnp.float32)
    b_vec = jnp.broadcast_to(b.reshape(1), (L,)).astype(jnp.float32)
    out = _gmf(user_indices.astype(jnp.int32), item_indices.astype(jnp.int32),
               user_table.T, item_table.T, w_flat, b_vec, B=B, D=D)
    return out.reshape(B, 1)
